# baseline (speedup 1.0000x reference)
import functools

import jax
import jax.numpy as jnp
from jax import lax
from jax.experimental import pallas as pl
from jax.experimental.pallas import tpu as pltpu
from jax.experimental.pallas import tpu_sc as plsc

_B = 16384
_D = 128
_L = 16
_C = 32
_NSETS = 4
_NC = 2
_NS = 16
_NW = _NC * _NS


def _vsqrt(x):
    xc = jnp.maximum(x, jnp.float32(1e-30))
    bits = plsc.bitcast(xc, jnp.int32)
    seed = plsc.bitcast((bits >> 1) + jnp.int32(0x1FBD1DF5), jnp.float32)
    y = seed
    for _ in range(3):
        y = jnp.float32(0.5) * (y + xc / y)
    return y


@functools.lru_cache(maxsize=1)
def _build():
    bpw = _B // _NW
    nch = bpw // _C
    nsl = _D // _L
    mesh = plsc.VectorSubcoreMesh(
        core_axis_name="c", subcore_axis_name="s",
        num_cores=_NC, num_subcores=_NS)

    def _chunk_scratch():
        return [
            pltpu.VMEM((_C, _D), jnp.float32),
            pltpu.VMEM((_C, _D), jnp.float32),
            pltpu.VMEM((_C, _D), jnp.float32),
            pltpu.VMEM((_C, _D), jnp.float32),
            pltpu.VMEM((_C, _D), jnp.float32),
            pltpu.VMEM((_C, _D), jnp.float32),
            pltpu.VMEM((_C,), jnp.float32),
            pltpu.SemaphoreType.DMA,
            pltpu.SemaphoreType.DMA,
        ]

    @functools.partial(
        pl.kernel,
        mesh=mesh,
        out_type=jax.ShapeDtypeStruct((_B,), jnp.float32),
        scratch_types=[
            [_chunk_scratch() for _ in range(_NSETS)],
            pltpu.VMEM((bpw,), jnp.int32),
            pltpu.VMEM((bpw,), jnp.int32),
            pltpu.VMEM((bpw,), jnp.int32),
            pltpu.SemaphoreType.DMA,
        ],
        compiler_params=pltpu.CompilerParams(needs_layout_passes=False),
    )
    def trans_d(heads, rels, tails, ee, ep, ret, rpt, out,
                sets, idxh, idxr, idxt, isem):
        wid = lax.axis_index("s") * _NC + lax.axis_index("c")
        base = wid * bpw
        lane = lax.iota(jnp.int32, _L)

        pltpu.async_copy(heads.at[pl.ds(base, bpw)], idxh, isem)
        pltpu.async_copy(rels.at[pl.ds(base, bpw)], idxr, isem)
        pltpu.async_copy(tails.at[pl.ds(base, bpw)], idxt, isem)
        for src, dst in ((heads, idxh), (rels, idxr), (tails, idxt)):
            pltpu.make_async_copy(src.at[pl.ds(base, bpw)], dst, isem).wait()

        def issue(buf, loc):
            he, hp, te, tp, re, rp, scores, sem, osem = buf
            ih = idxh.at[pl.ds(loc, _C)]
            it_ = idxt.at[pl.ds(loc, _C)]
            ir = idxr.at[pl.ds(loc, _C)]
            pltpu.async_copy(ee.at[ih], he, sem)
            pltpu.async_copy(ep.at[ih], hp, sem)
            pltpu.async_copy(ee.at[it_], te, sem)
            pltpu.async_copy(ep.at[it_], tp, sem)
            pltpu.async_copy(ret.at[ir], re, sem)
            pltpu.async_copy(rpt.at[ir], rp, sem)

        def drain(buf):
            he, hp, te, tp, re, rp, scores, sem, osem = buf
            for dst in (he, hp, te, tp, re, rp):
                pltpu.make_async_copy(ee.at[pl.ds(0, _C)], dst, sem).wait()

        def drain_out(buf):
            he, hp, te, tp, re, rp, scores, sem, osem = buf
            pltpu.make_async_copy(out.at[pl.ds(base, _C)], scores, osem).wait()

        def compute(buf, loc):
            he, hp, te, tp, re, rp, scores, sem, osem = buf

            def group_body(g, carry2):
                vec = jnp.zeros((_L,), jnp.float32)
                for k in range(_L):
                    i = g * _L + k
                    sl = pl.ds(0, _L)
                    acch = hp[i, sl] * he[i, sl]
                    acct = tp[i, sl] * te[i, sl]
                    for j in range(1, nsl):
                        sl = pl.ds(j * _L, _L)
                        acch = acch + hp[i, sl] * he[i, sl]
                        acct = acct + tp[i, sl] * te[i, sl]
                    s = jnp.sum(acch) - jnp.sum(acct)
                    sl = pl.ds(0, _L)
                    v = he[i, sl] - te[i, sl] + re[i, sl] + s * rp[i, sl]
                    nsq = v * v
                    for j in range(1, nsl):
                        sl = pl.ds(j * _L, _L)
                        v = he[i, sl] - te[i, sl] + re[i, sl] + s * rp[i, sl]
                        nsq = nsq + v * v
                    vec = jnp.where(lane == k, jnp.sum(nsq), vec)
                scores[pl.ds(g * _L, _L)] = _vsqrt(vec)
                return carry2

            lax.fori_loop(0, _C // _L, group_body, 0)
            pltpu.async_copy(scores, out.at[pl.ds(base + loc, _C)], osem)

        issue(sets[0], 0)
        issue(sets[1], _C)

        def ring_body(m, carry):
            loc0 = (_NSETS * m) * _C
            for q in range(_NSETS):
                loc = loc0 + q * _C
                drain(sets[q])

                @pl.when(loc + 2 * _C < bpw)
                def _():
                    issue(sets[(q + 2) % _NSETS], loc + 2 * _C)

                @pl.when(m > 0)
                def _():
                    drain_out(sets[q])

                compute(sets[q], loc)
            return carry

        lax.fori_loop(0, nch // _NSETS, ring_body, 0)
        for q in range(_NSETS):
            drain_out(sets[q])

    return trans_d


def kernel(heads, relations, tails, entity_embeddings, entity_projections,
           relation_embeddings, relation_projections):
    k = _build()
    return k(
        heads.astype(jnp.int32),
        relations.astype(jnp.int32),
        tails.astype(jnp.int32),
        entity_embeddings,
        entity_projections,
        relation_embeddings,
        relation_projections,
    )


# speedup vs baseline: 1.2397x; 1.2397x over previous
import functools

import jax
import jax.numpy as jnp
from jax import lax
from jax.experimental import pallas as pl
from jax.experimental.pallas import tpu as pltpu
from jax.experimental.pallas import tpu_sc as plsc

_B = 16384
_D = 128
_L = 16
_C = 64
_NC = 2
_NS = 16
_NW = _NC * _NS


def _vsqrt(x):
    xc = jnp.maximum(x, jnp.float32(1e-30))
    bits = plsc.bitcast(xc, jnp.int32)
    seed = plsc.bitcast((bits >> 1) + jnp.int32(0x1FBD1DF5), jnp.float32)
    y = seed
    for _ in range(3):
        y = jnp.float32(0.5) * (y + xc / y)
    return y


@functools.lru_cache(maxsize=1)
def _build():
    bpw = _B // _NW
    nch = bpw // _C
    nsl = _D // _L
    mesh = plsc.VectorSubcoreMesh(
        core_axis_name="c", subcore_axis_name="s",
        num_cores=_NC, num_subcores=_NS)

    def _chunk_scratch():
        return [
            pltpu.VMEM((_C, _D), jnp.float32),
            pltpu.VMEM((_C, _D), jnp.float32),
            pltpu.VMEM((_C, _D), jnp.float32),
            pltpu.VMEM((_C, _D), jnp.float32),
            pltpu.VMEM((_C, _D), jnp.float32),
            pltpu.VMEM((_C, _D), jnp.float32),
            pltpu.VMEM((_C,), jnp.float32),
            pltpu.SemaphoreType.DMA,
            pltpu.SemaphoreType.DMA,
        ]

    @functools.partial(
        pl.kernel,
        mesh=mesh,
        out_type=jax.ShapeDtypeStruct((_B,), jnp.float32),
        scratch_types=[
            _chunk_scratch(),
            _chunk_scratch(),
            pltpu.VMEM((bpw,), jnp.int32),
            pltpu.VMEM((bpw,), jnp.int32),
            pltpu.VMEM((bpw,), jnp.int32),
        ],
        compiler_params=pltpu.CompilerParams(needs_layout_passes=False),
    )
    def trans_d(heads, rels, tails, ee, ep, ret, rpt, out,
                set0, set1, idxh, idxr, idxt):
        wid = lax.axis_index("s") * _NC + lax.axis_index("c")
        base = wid * bpw
        lane = lax.iota(jnp.int32, _L)
        sets = (set0, set1)

        pltpu.sync_copy(heads.at[pl.ds(base, bpw)], idxh)
        pltpu.sync_copy(rels.at[pl.ds(base, bpw)], idxr)
        pltpu.sync_copy(tails.at[pl.ds(base, bpw)], idxt)

        def issue(buf, loc):
            he, hp, te, tp, re, rp, scores, sem, osem = buf
            ih = idxh.at[pl.ds(loc, _C)]
            it_ = idxt.at[pl.ds(loc, _C)]
            ir = idxr.at[pl.ds(loc, _C)]
            pltpu.async_copy(ee.at[ih], he, sem)
            pltpu.async_copy(ep.at[ih], hp, sem)
            pltpu.async_copy(ee.at[it_], te, sem)
            pltpu.async_copy(ep.at[it_], tp, sem)
            pltpu.async_copy(ret.at[ir], re, sem)
            pltpu.async_copy(rpt.at[ir], rp, sem)

        def drain(buf):
            he, hp, te, tp, re, rp, scores, sem, osem = buf
            for dst in (he, hp, te, tp, re, rp):
                pltpu.make_async_copy(ee.at[pl.ds(0, _C)], dst, sem).wait()

        def drain_out(buf):
            he, hp, te, tp, re, rp, scores, sem, osem = buf
            pltpu.make_async_copy(out.at[pl.ds(base, _C)], scores, osem).wait()

        def compute(buf, loc):
            he, hp, te, tp, re, rp, scores, sem, osem = buf

            def group_body(g, carry2):
                vec = jnp.zeros((_L,), jnp.float32)
                for k in range(_L):
                    i = g * _L + k
                    sl = pl.ds(0, _L)
                    acch = hp[i, sl] * he[i, sl]
                    acct = tp[i, sl] * te[i, sl]
                    for j in range(1, nsl):
                        sl = pl.ds(j * _L, _L)
                        acch = acch + hp[i, sl] * he[i, sl]
                        acct = acct + tp[i, sl] * te[i, sl]
                    s = jnp.sum(acch) - jnp.sum(acct)
                    sl = pl.ds(0, _L)
                    v = he[i, sl] - te[i, sl] + re[i, sl] + s * rp[i, sl]
                    nsq = v * v
                    for j in range(1, nsl):
                        sl = pl.ds(j * _L, _L)
                        v = he[i, sl] - te[i, sl] + re[i, sl] + s * rp[i, sl]
                        nsq = nsq + v * v
                    vec = jnp.where(lane == k, jnp.sum(nsq), vec)
                scores[pl.ds(g * _L, _L)] = _vsqrt(vec)
                return carry2

            lax.fori_loop(0, _C // _L, group_body, 0)
            pltpu.async_copy(scores, out.at[pl.ds(base + loc, _C)], osem)

        issue(sets[0], 0)

        def pair_body(m, carry):
            loc0 = (2 * m) * _C
            drain(sets[0])
            issue(sets[1], loc0 + _C)

            @pl.when(m > 0)
            def _():
                drain_out(sets[0])

            compute(sets[0], loc0)
            drain(sets[1])

            @pl.when(m < nch // 2 - 1)
            def _():
                issue(sets[0], loc0 + 2 * _C)

            @pl.when(m > 0)
            def _():
                drain_out(sets[1])

            compute(sets[1], loc0 + _C)
            return carry

        lax.fori_loop(0, nch // 2, pair_body, 0)
        drain_out(sets[0])
        drain_out(sets[1])

    return trans_d


def kernel(heads, relations, tails, entity_embeddings, entity_projections,
           relation_embeddings, relation_projections):
    k = _build()
    return k(
        heads.astype(jnp.int32),
        relations.astype(jnp.int32),
        tails.astype(jnp.int32),
        entity_embeddings,
        entity_projections,
        relation_embeddings,
        relation_projections,
    )
